# Initial kernel scaffold; baseline (speedup 1.0000x reference)
import jax
import jax.numpy as jnp
from jax.experimental import pallas as pl


def kernel(hidden_states, topk_weights, topk_ids, gate_up_weights, down_weights):
    raise NotImplementedError("write your pallas kernel here")


# speedup vs baseline: 3.6571x; 3.6571x over previous
import functools

import jax
import jax.numpy as jnp
from jax import lax
from jax.experimental import pallas as pl
from jax.experimental.pallas import tpu as pltpu

T = 2048
D = 768
F = 3072
E = 8
K = 2
BLOCK_M = 256
NT = (T * K) // BLOCK_M + E
M_PAD = NT * BLOCK_M


def _gu_body(eot_ref, rows_ref, x_ref, wg_ref, wu_ref, g_ref):
    m = pl.program_id(0)

    @pl.when(rows_ref[m] > 0)
    def _():
        x = x_ref[...]
        hg = lax.dot_general(x, wg_ref[0], (((1,), (1,)), ((), ())),
                             preferred_element_type=jnp.float32)
        hu = lax.dot_general(x, wu_ref[0], (((1,), (1,)), ((), ())),
                             preferred_element_type=jnp.float32)
        g_ref[...] = hg * jax.nn.sigmoid(hg) * hu


_grouped_gu = pl.pallas_call(
    _gu_body,
    grid_spec=pltpu.PrefetchScalarGridSpec(
        num_scalar_prefetch=2,
        grid=(NT,),
        in_specs=[
            pl.BlockSpec((BLOCK_M, D), lambda m, eot, rows: (m, 0)),
            pl.BlockSpec((1, F, D), lambda m, eot, rows: (eot[m], 0, 0)),
            pl.BlockSpec((1, F, D), lambda m, eot, rows: (eot[m], 1, 0)),
        ],
        out_specs=pl.BlockSpec((BLOCK_M, F), lambda m, eot, rows: (m, 0)),
    ),
    out_shape=jax.ShapeDtypeStruct((M_PAD, F), jnp.float32),
)


def _down_body(eot_ref, rows_ref, g_ref, wd_ref, ws_ref, o_ref):
    m = pl.program_id(0)

    @pl.when(rows_ref[m] > 0)
    def _():
        o = lax.dot_general(g_ref[...], wd_ref[0], (((1,), (1,)), ((), ())),
                            preferred_element_type=jnp.float32)
        o_ref[...] = o * ws_ref[0, 0, :][:, None]


_grouped_down = pl.pallas_call(
    _down_body,
    grid_spec=pltpu.PrefetchScalarGridSpec(
        num_scalar_prefetch=2,
        grid=(NT,),
        in_specs=[
            pl.BlockSpec((BLOCK_M, F), lambda m, eot, rows: (m, 0)),
            pl.BlockSpec((1, D, F), lambda m, eot, rows: (eot[m], 0, 0)),
            pl.BlockSpec((1, 1, BLOCK_M), lambda m, eot, rows: (m, 0, 0)),
        ],
        out_specs=pl.BlockSpec((BLOCK_M, D), lambda m, eot, rows: (m, 0)),
    ),
    out_shape=jax.ShapeDtypeStruct((M_PAD, D), jnp.float32),
)


def kernel(hidden_states, topk_weights, topk_ids, gate_up_weights, down_weights):
    flat_ids = topk_ids.reshape(-1).astype(jnp.int32)
    flat_w = topk_weights.reshape(-1)
    tok_of_slot = (jnp.arange(T * K, dtype=jnp.int32) // K)

    onehot = (flat_ids[:, None] == jnp.arange(E, dtype=jnp.int32)[None, :])
    csum = jnp.cumsum(onehot.astype(jnp.int32), axis=0)
    counts = csum[-1]
    rank = jnp.sum(jnp.where(onehot, csum - 1, 0), axis=1)

    tiles_per_e = (counts + BLOCK_M - 1) // BLOCK_M
    cum_tiles = jnp.cumsum(tiles_per_e)
    tile_off_e = cum_tiles - tiles_per_e
    pos = tile_off_e[flat_ids] * BLOCK_M + rank

    src_tok = jnp.zeros((M_PAD,), jnp.int32).at[pos].set(tok_of_slot)
    ws = jnp.zeros((M_PAD,), jnp.float32).at[pos].set(flat_w)

    tile_idx = jnp.arange(NT, dtype=jnp.int32)
    eot = jnp.sum(tile_idx[:, None] >= cum_tiles[None, :], axis=1)
    eot = jnp.minimum(eot, E - 1).astype(jnp.int32)
    tile_in_e = tile_idx - tile_off_e[eot]
    rows = jnp.clip(counts[eot] - tile_in_e * BLOCK_M, 0, BLOCK_M).astype(jnp.int32)

    x_pad = hidden_states[src_tok]
    g = _grouped_gu(eot, rows, x_pad, gate_up_weights, gate_up_weights)
    y = _grouped_down(eot, rows, g, down_weights, ws.reshape(NT, 1, BLOCK_M))
    posk = pos.reshape(T, K)
    out = y[posk[:, 0]] + y[posk[:, 1]]
    return out
